# baseline (speedup 1.0000x reference)
import functools

import jax
import jax.numpy as jnp
from jax import lax
from jax.experimental import pallas as pl
from jax.experimental.pallas import tpu as pltpu
from jax.experimental.pallas import tpu_sc as plsc

_BINS = (1, 2, 3, 4, 8, 16, 32, 64)
_NUM_EMB = 9
_DIM = 20
_N = 16384

_NC = 2
_NS = 16
_NW = _NC * _NS
_BPW = _N // _NW
_L = 16
_TAB_PAD = 256
_NCHUNK = 4
_RPC = _BPW // _NCHUNK
_GPC = _BPW // _L // _NCHUNK


def _sc_body(len_hbm, tab_hbm, out_hbm, len_v, tab_v, out_v, sem, isem):
    wid = lax.axis_index("s") * _NC + lax.axis_index("c")
    base = wid * _BPW

    for j in range(_NCHUNK):
        pltpu.async_copy(
            len_hbm.at[pl.ds(base + j * _RPC, _RPC)],
            len_v.at[pl.ds(j * _RPC, _RPC)],
            isem,
        )
    in2 = pltpu.async_copy(tab_hbm, tab_v.at[pl.ds(0, _NUM_EMB * _DIM)], sem)
    in2.wait()

    lane = lax.iota(jnp.int32, _L)

    def _chunk(j, carry):
        pltpu.make_async_copy(
            len_hbm.at[pl.ds(base + j * _RPC, _RPC)],
            len_v.at[pl.ds(j * _RPC, _RPC)],
            isem,
        ).wait()

        @plsc.parallel_loop(j * _GPC, (j + 1) * _GPC, step=1, unroll=2)
        def _group(g):
            lens = len_v[pl.ds(g * _L, _L)]
            acc = jnp.zeros((_L,), jnp.int32)
            for b in _BINS:
                acc = acc + jnp.where(lens > b, 1, 0).astype(jnp.int32)
            pos = acc * _DIM
            elem = lane + g * _L
            for d in range(_DIM):
                vals = plsc.load_gather(tab_v, [pos + d])
                plsc.store_scatter(out_v, [elem, jnp.full((_L,), d, jnp.int32)], vals)

        pltpu.async_copy(
            out_v.at[pl.ds(j * _RPC, _RPC)],
            out_hbm.at[pl.ds(base + j * _RPC, _RPC)],
            sem,
        )
        return carry

    lax.fori_loop(0, _NCHUNK, _chunk, 0)
    for j in range(_NCHUNK):
        pltpu.make_async_copy(
            out_v.at[pl.ds(j * _RPC, _RPC)],
            out_hbm.at[pl.ds(base + j * _RPC, _RPC)],
            sem,
        ).wait()


@functools.partial(
    pl.kernel,
    out_type=jax.ShapeDtypeStruct((_N, _DIM), jnp.float32),
    mesh=plsc.VectorSubcoreMesh(core_axis_name="c", subcore_axis_name="s"),
    compiler_params=pltpu.CompilerParams(needs_layout_passes=False),
    scratch_types=[
        pltpu.VMEM((_BPW,), jnp.int32),
        pltpu.VMEM((_TAB_PAD,), jnp.float32),
        pltpu.VMEM((_BPW, _DIM), jnp.float32),
        pltpu.SemaphoreType.DMA,
        pltpu.SemaphoreType.DMA,
    ],
)
def _sc_lookup(len_hbm, tab_hbm, out_hbm, len_v, tab_v, out_v, sem, isem):
    _sc_body(len_hbm, tab_hbm, out_hbm, len_v, tab_v, out_v, sem, isem)


def kernel(lengths, table):
    return _sc_lookup(lengths.astype(jnp.int32), table.reshape(-1))


# speedup vs baseline: 1.0005x; 1.0005x over previous
import functools

import jax
import jax.numpy as jnp
from jax import lax
from jax.experimental import pallas as pl
from jax.experimental.pallas import tpu as pltpu
from jax.experimental.pallas import tpu_sc as plsc

_BINS = (1, 2, 3, 4, 8, 16, 32, 64)
_NUM_EMB = 9
_DIM = 20
_N = 16384

_NC = 2
_NS = 16
_NW = _NC * _NS
_BPW = _N // _NW
_L = 16
_TAB_PAD = 256
_NCHUNK = 4
_RPC = _BPW // _NCHUNK
_GPC = _BPW // _L // _NCHUNK


def _sc_body(len_hbm, tab_hbm, out_hbm, len_v, tab_v, out_v, sem, isem):
    wid = lax.axis_index("s") * _NC + lax.axis_index("c")
    base = wid * _BPW

    in1 = pltpu.async_copy(len_hbm.at[pl.ds(base, _BPW)], len_v, isem)
    in2 = pltpu.async_copy(tab_hbm, tab_v.at[pl.ds(0, _NUM_EMB * _DIM)], sem)
    in1.wait()
    in2.wait()

    lane = lax.iota(jnp.int32, _L)

    def _chunk(j, carry):
        @plsc.parallel_loop(j * _GPC, (j + 1) * _GPC, step=1, unroll=2)
        def _group(g):
            lens = len_v[pl.ds(g * _L, _L)]
            acc = jnp.zeros((_L,), jnp.int32)
            for b in _BINS:
                acc = acc + jnp.where(lens > b, 1, 0).astype(jnp.int32)
            pos = acc * _DIM
            elem = lane + g * _L
            for d in range(_DIM):
                vals = plsc.load_gather(tab_v, [pos + d])
                plsc.store_scatter(out_v, [elem, jnp.full((_L,), d, jnp.int32)], vals)

        pltpu.async_copy(
            out_v.at[pl.ds(j * _RPC, _RPC)],
            out_hbm.at[pl.ds(base + j * _RPC, _RPC)],
            sem,
        )
        return carry

    lax.fori_loop(0, _NCHUNK, _chunk, 0)
    for j in range(_NCHUNK):
        pltpu.make_async_copy(
            out_v.at[pl.ds(j * _RPC, _RPC)],
            out_hbm.at[pl.ds(base + j * _RPC, _RPC)],
            sem,
        ).wait()


@functools.partial(
    pl.kernel,
    out_type=jax.ShapeDtypeStruct((_N, _DIM), jnp.float32),
    mesh=plsc.VectorSubcoreMesh(core_axis_name="c", subcore_axis_name="s"),
    compiler_params=pltpu.CompilerParams(needs_layout_passes=False),
    scratch_types=[
        pltpu.VMEM((_BPW,), jnp.int32),
        pltpu.VMEM((_TAB_PAD,), jnp.float32),
        pltpu.VMEM((_BPW, _DIM), jnp.float32),
        pltpu.SemaphoreType.DMA,
        pltpu.SemaphoreType.DMA,
    ],
)
def _sc_lookup(len_hbm, tab_hbm, out_hbm, len_v, tab_v, out_v, sem, isem):
    _sc_body(len_hbm, tab_hbm, out_hbm, len_v, tab_v, out_v, sem, isem)


def kernel(lengths, table):
    return _sc_lookup(lengths.astype(jnp.int32), table.reshape(-1))
